# baseline (speedup 1.0000x reference)
import functools

import jax
import jax.numpy as jnp
from jax import lax
from jax.experimental import pallas as pl
from jax.experimental.pallas import tpu as pltpu
from jax.experimental.pallas import tpu_sc as plsc

MODEL_DIM = 2048
NUM_EXPERTS = 16
NUM_TOKENS = 8192
BLOCK_T = 1024

_NS = 16
_CHUNK = NUM_TOKENS // _NS
_L = 16


def _gate_body(x1_ref, x2_ref, w_ref, idx_ref, gate_ref, laux_ref,
               cnt_ref, me_ref):
    pid = pl.program_id(0)
    nblk = pl.num_programs(0)

    @pl.when(pid == 0)
    def _init():
        cnt_ref[...] = jnp.zeros_like(cnt_ref)
        me_ref[...] = jnp.zeros_like(me_ref)

    w = w_ref[...]
    lg1 = lax.dot_general(w, x1_ref[...], (((1,), (1,)), ((), ())),
                          preferred_element_type=jnp.float32)
    lg2 = lax.dot_general(w, x2_ref[...], (((1,), (1,)), ((), ())),
                          preferred_element_type=jnp.float32)
    lg = jnp.concatenate([lg1, lg2], axis=1)

    m = jnp.max(lg, axis=0, keepdims=True)
    p = jnp.exp(lg - m)
    s = jnp.sum(p, axis=0, keepdims=True)
    inv_s = 1.0 / s
    gate_ref[...] = inv_s[0]

    si = lax.broadcasted_iota(jnp.int32, lg.shape, 0)
    eq = lg == m
    idx = jnp.min(jnp.where(eq, si, NUM_EXPERTS), axis=0)
    idx_ref[...] = idx.astype(jnp.int32)

    mask = (si == idx[None, :]).astype(jnp.float32)

    me_ref[...] = me_ref[...] + jnp.sum(p * inv_s, axis=1, keepdims=True)
    cnt_ref[...] = cnt_ref[...] + jnp.sum(mask, axis=1, keepdims=True)

    @pl.when(pid == nblk - 1)
    def _fin():
        prod = me_ref[...] * cnt_ref[...]
        laux_ref[...] = jnp.sum(prod, axis=0, keepdims=True) * (
            NUM_EXPERTS / (NUM_TOKENS * NUM_TOKENS))


def _loc_sc_body(idx_hbm, out_hbm, idx_v, loc_v, run_ref, off_ref,
                 tmp_ref, pub_ref, hists_v, shared):
    wid = lax.axis_index("s")
    base = wid * _CHUNK
    pltpu.sync_copy(idx_hbm.at[pl.ds(base, _CHUNK)], idx_v)

    run_ref[...] = jnp.zeros((_L,), jnp.int32)
    lane = lax.broadcasted_iota(jnp.int32, (_L,), 0)
    one = jnp.ones((_L,), jnp.int32)
    zero = jnp.zeros((_L,), jnp.int32)
    tmp_ref[pl.ds(0, _L)] = jnp.full((_L,), -1, jnp.int32)
    tmp_ref[pl.ds(2 * _L, _L)] = jnp.full((_L,), -2, jnp.int32)
    back_idx = [lane + (_L - t) for t in range(1, _L)]
    fwd_idx = [lane + (_L + t) for t in range(1, _L)]
    for j in range(_CHUNK // _L):
        v = idx_v[pl.ds(j * _L, _L)]
        tmp_ref[pl.ds(_L, _L)] = v
        excl = zero
        fwd = zero
        for t in range(_L - 1):
            vb = plsc.load_gather(tmp_ref, [back_idx[t]])
            excl = excl + jnp.where(vb == v, one, zero)
            vf = plsc.load_gather(tmp_ref, [fwd_idx[t]])
            fwd = fwd + jnp.where(vf == v, one, zero)
        g = plsc.load_gather(run_ref, [v])
        loc_v[pl.ds(j * _L, _L)] = g + excl
        plsc.addupdate_scatter(run_ref, [v], excl + one, mask=(fwd == 0))

    hist_final = plsc.load_gather(run_ref, [lane])
    pub_ref[...] = hist_final
    pltpu.sync_copy(pub_ref, shared.at[wid, pl.ds(0, _L)])
    plsc.subcore_barrier()
    off = jnp.zeros((_L,), jnp.int32)
    widv = jnp.full((_L,), wid, jnp.int32)
    for w in range(_NS):
        pltpu.sync_copy(shared.at[w, pl.ds(0, _L)], hists_v.at[w])
        row = hists_v[w]
        off = off + jnp.where(jnp.full((_L,), w, jnp.int32) < widv, row, 0)
    off_ref[...] = off

    for j in range(_CHUNK // _L):
        v = idx_v[pl.ds(j * _L, _L)]
        loc_v[pl.ds(j * _L, _L)] = (loc_v[pl.ds(j * _L, _L)]
                                    + plsc.load_gather(off_ref, [v]))

    pltpu.sync_copy(loc_v, out_hbm.at[pl.ds(base, _CHUNK)])


@functools.partial(
    pl.kernel,
    out_type=jax.ShapeDtypeStruct((NUM_TOKENS,), jnp.int32),
    mesh=plsc.VectorSubcoreMesh(core_axis_name="c", subcore_axis_name="s",
                                num_cores=1),
    scratch_types=[
        pltpu.VMEM((_CHUNK,), jnp.int32),
        pltpu.VMEM((_CHUNK,), jnp.int32),
        pltpu.VMEM((_L,), jnp.int32),
        pltpu.VMEM((_L,), jnp.int32),
        pltpu.VMEM((3 * _L,), jnp.int32),
        pltpu.VMEM((_L,), jnp.int32),
        pltpu.VMEM((_NS, _L), jnp.int32),
        pltpu.VMEM_SHARED((_NS, 128), jnp.int32),
    ],
    compiler_params=pltpu.CompilerParams(needs_layout_passes=False),
)
def _locations_sc(idx_hbm, out_hbm, idx_v, loc_v, run_ref, off_ref,
                  tmp_ref, pub_ref, hists_v, shared):
    _loc_sc_body(idx_hbm, out_hbm, idx_v, loc_v, run_ref, off_ref,
                 tmp_ref, pub_ref, hists_v, shared)


@jax.jit
def _top1_gate(x, W):
    nblk = NUM_TOKENS // BLOCK_T
    out_shapes = (
        jax.ShapeDtypeStruct((NUM_TOKENS,), jnp.int32),
        jax.ShapeDtypeStruct((NUM_TOKENS,), jnp.float32),
        jax.ShapeDtypeStruct((1, 1), jnp.float32),
    )
    out = pl.pallas_call(
        _gate_body,
        grid=(nblk,),
        in_specs=[
            pl.BlockSpec((BLOCK_T // 2, MODEL_DIM), lambda i: (2 * i, 0)),
            pl.BlockSpec((BLOCK_T // 2, MODEL_DIM), lambda i: (2 * i + 1, 0)),
            pl.BlockSpec((NUM_EXPERTS, MODEL_DIM), lambda i: (0, 0)),
        ],
        out_specs=(
            pl.BlockSpec((BLOCK_T,), lambda i: (i,)),
            pl.BlockSpec((BLOCK_T,), lambda i: (i,)),
            pl.BlockSpec((1, 1), lambda i: (0, 0)),
        ),
        out_shape=out_shapes,
        scratch_shapes=[
            pltpu.VMEM((NUM_EXPERTS, 1), jnp.float32),
            pltpu.VMEM((NUM_EXPERTS, 1), jnp.float32),
        ],
        compiler_params=pltpu.CompilerParams(
            dimension_semantics=("arbitrary",),
        ),
    )(x, x, W)
    idx, gates1, laux = out
    loc = _locations_sc(idx)
    return laux[0, 0], idx, loc, gates1


def kernel(x, W):
    laux, idx, loc, gates1 = _top1_gate(x, W)
    capacity = (NUM_TOKENS + NUM_EXPERTS - 1) // NUM_EXPERTS
    return (laux, idx, capacity, loc, gates1, NUM_EXPERTS)


# speedup vs baseline: 1.0571x; 1.0571x over previous
import functools

import jax
import jax.numpy as jnp
from jax import lax
from jax.experimental import pallas as pl
from jax.experimental.pallas import tpu as pltpu
from jax.experimental.pallas import tpu_sc as plsc

MODEL_DIM = 2048
NUM_EXPERTS = 16
NUM_TOKENS = 8192
BLOCK_T = 1024

_NS = 16
_CHUNK = NUM_TOKENS // _NS
_L = 16


def _gate_body(x1_ref, x2_ref, w_ref, idx_ref, gate_ref, laux_ref,
               cnt_ref, me_ref):
    pid = pl.program_id(0)
    nblk = pl.num_programs(0)

    @pl.when(pid == 0)
    def _init():
        cnt_ref[...] = jnp.zeros_like(cnt_ref)
        me_ref[...] = jnp.zeros_like(me_ref)

    w = w_ref[...]
    lg1 = lax.dot_general(w, x1_ref[...], (((1,), (1,)), ((), ())),
                          preferred_element_type=jnp.float32)
    lg2 = lax.dot_general(w, x2_ref[...], (((1,), (1,)), ((), ())),
                          preferred_element_type=jnp.float32)
    lg = jnp.concatenate([lg1, lg2], axis=1)

    m = jnp.max(lg, axis=0, keepdims=True)
    p = jnp.exp(lg - m)
    s = jnp.sum(p, axis=0, keepdims=True)
    inv_s = 1.0 / s
    gate_ref[...] = inv_s[0]

    si = lax.broadcasted_iota(jnp.int32, lg.shape, 0)
    eq = lg == m
    idx = jnp.min(jnp.where(eq, si, NUM_EXPERTS), axis=0)
    idx_ref[...] = idx.astype(jnp.int32)

    mask = (si == idx[None, :]).astype(jnp.float32)

    me_ref[...] = me_ref[...] + jnp.sum(p * inv_s, axis=1, keepdims=True)
    cnt_ref[...] = cnt_ref[...] + jnp.sum(mask, axis=1, keepdims=True)

    @pl.when(pid == nblk - 1)
    def _fin():
        prod = me_ref[...] * cnt_ref[...]
        laux_ref[...] = jnp.sum(prod, axis=0, keepdims=True) * (
            NUM_EXPERTS / (NUM_TOKENS * NUM_TOKENS))


def _loc_sc_body(idx_hbm, out_hbm, idx_v, loc_v, run_ref, off_ref,
                 tmp_ref, pub_ref, hists_v, shared):
    wid = lax.axis_index("s")
    base = wid * _CHUNK
    pltpu.sync_copy(idx_hbm.at[pl.ds(base, _CHUNK)], idx_v)

    run_ref[...] = jnp.zeros((_L,), jnp.int32)
    lane = lax.broadcasted_iota(jnp.int32, (_L,), 0)
    one = jnp.ones((_L,), jnp.int32)
    zero = jnp.zeros((_L,), jnp.int32)
    tmp_ref[pl.ds(0, _L)] = jnp.full((_L,), -1, jnp.int32)
    tmp_ref[pl.ds(2 * _L, _L)] = jnp.full((_L,), -2, jnp.int32)
    back_idx = [lane + (_L - t) for t in range(1, _L)]
    fwd_idx = [lane + (_L + t) for t in range(1, _L)]
    for j in range(_CHUNK // _L):
        v = idx_v[pl.ds(j * _L, _L)]
        tmp_ref[pl.ds(_L, _L)] = v
        excl = zero
        for t in range(_L - 1):
            vb = plsc.load_gather(tmp_ref, [back_idx[t]])
            excl = excl + jnp.where(vb == v, one, zero)
        g = plsc.load_gather(run_ref, [v])
        loc_v[pl.ds(j * _L, _L)] = g + excl
        plsc.addupdate_scatter(run_ref, [v], one)

    hist_final = plsc.load_gather(run_ref, [lane])
    pub_ref[...] = hist_final
    pltpu.sync_copy(pub_ref, shared.at[wid, pl.ds(0, _L)])
    plsc.subcore_barrier()
    pltpu.sync_copy(shared, hists_v)
    off = jnp.zeros((_L,), jnp.int32)
    widv = jnp.full((_L,), wid, jnp.int32)
    for w in range(_NS):
        row = hists_v[w, pl.ds(0, _L)]
        off = off + jnp.where(jnp.full((_L,), w, jnp.int32) < widv, row, 0)
    off_ref[...] = off

    for j in range(_CHUNK // _L):
        v = idx_v[pl.ds(j * _L, _L)]
        loc_v[pl.ds(j * _L, _L)] = (loc_v[pl.ds(j * _L, _L)]
                                    + plsc.load_gather(off_ref, [v]))

    pltpu.sync_copy(loc_v, out_hbm.at[pl.ds(base, _CHUNK)])


@functools.partial(
    pl.kernel,
    out_type=jax.ShapeDtypeStruct((NUM_TOKENS,), jnp.int32),
    mesh=plsc.VectorSubcoreMesh(core_axis_name="c", subcore_axis_name="s",
                                num_cores=1),
    scratch_types=[
        pltpu.VMEM((_CHUNK,), jnp.int32),
        pltpu.VMEM((_CHUNK,), jnp.int32),
        pltpu.VMEM((_L,), jnp.int32),
        pltpu.VMEM((_L,), jnp.int32),
        pltpu.VMEM((3 * _L,), jnp.int32),
        pltpu.VMEM((_L,), jnp.int32),
        pltpu.VMEM((_NS, 128), jnp.int32),
        pltpu.VMEM_SHARED((_NS, 128), jnp.int32),
    ],
    compiler_params=pltpu.CompilerParams(needs_layout_passes=False),
)
def _locations_sc(idx_hbm, out_hbm, idx_v, loc_v, run_ref, off_ref,
                  tmp_ref, pub_ref, hists_v, shared):
    _loc_sc_body(idx_hbm, out_hbm, idx_v, loc_v, run_ref, off_ref,
                 tmp_ref, pub_ref, hists_v, shared)


@jax.jit
def _top1_gate(x, W):
    nblk = NUM_TOKENS // BLOCK_T
    out_shapes = (
        jax.ShapeDtypeStruct((NUM_TOKENS,), jnp.int32),
        jax.ShapeDtypeStruct((NUM_TOKENS,), jnp.float32),
        jax.ShapeDtypeStruct((1, 1), jnp.float32),
    )
    out = pl.pallas_call(
        _gate_body,
        grid=(nblk,),
        in_specs=[
            pl.BlockSpec((BLOCK_T // 2, MODEL_DIM), lambda i: (2 * i, 0)),
            pl.BlockSpec((BLOCK_T // 2, MODEL_DIM), lambda i: (2 * i + 1, 0)),
            pl.BlockSpec((NUM_EXPERTS, MODEL_DIM), lambda i: (0, 0)),
        ],
        out_specs=(
            pl.BlockSpec((BLOCK_T,), lambda i: (i,)),
            pl.BlockSpec((BLOCK_T,), lambda i: (i,)),
            pl.BlockSpec((1, 1), lambda i: (0, 0)),
        ),
        out_shape=out_shapes,
        scratch_shapes=[
            pltpu.VMEM((NUM_EXPERTS, 1), jnp.float32),
            pltpu.VMEM((NUM_EXPERTS, 1), jnp.float32),
        ],
        compiler_params=pltpu.CompilerParams(
            dimension_semantics=("arbitrary",),
        ),
    )(x, x, W)
    idx, gates1, laux = out
    loc = _locations_sc(idx)
    return laux[0, 0], idx, loc, gates1


def kernel(x, W):
    laux, idx, loc, gates1 = _top1_gate(x, W)
    capacity = (NUM_TOKENS + NUM_EXPERTS - 1) // NUM_EXPERTS
    return (laux, idx, capacity, loc, gates1, NUM_EXPERTS)


# speedup vs baseline: 1.6809x; 1.5901x over previous
import functools

import jax
import jax.numpy as jnp
from jax import lax
from jax.experimental import pallas as pl
from jax.experimental.pallas import tpu as pltpu

MODEL_DIM = 2048
NUM_EXPERTS = 16
NUM_TOKENS = 8192
BLOCK_T = 1024


def _gate_body(x1_ref, x2_ref, w_ref, idx_ref, loc_ref, gate_ref, laux_ref,
               cnt_ref, me_ref):
    pid = pl.program_id(0)
    nblk = pl.num_programs(0)

    @pl.when(pid == 0)
    def _init():
        cnt_ref[...] = jnp.zeros_like(cnt_ref)
        me_ref[...] = jnp.zeros_like(me_ref)

    w = w_ref[...]
    lg1 = lax.dot_general(w, x1_ref[...], (((1,), (1,)), ((), ())),
                          preferred_element_type=jnp.float32)
    lg2 = lax.dot_general(w, x2_ref[...], (((1,), (1,)), ((), ())),
                          preferred_element_type=jnp.float32)
    lg = jnp.concatenate([lg1, lg2], axis=1)

    m = jnp.max(lg, axis=0, keepdims=True)
    p = jnp.exp(lg - m)
    s = jnp.sum(p, axis=0, keepdims=True)
    inv_s = 1.0 / s
    gate_ref[...] = inv_s[0]

    si = lax.broadcasted_iota(jnp.int32, lg.shape, 0)
    eq = lg == m
    idx = jnp.min(jnp.where(eq, si, NUM_EXPERTS), axis=0)
    idx_ref[...] = idx.astype(jnp.int32)

    mask = (si == idx[None, :]).astype(jnp.float32)

    me_ref[...] = me_ref[...] + jnp.sum(p * inv_s, axis=1, keepdims=True)
    blk_cnt = jnp.sum(mask, axis=1, keepdims=True)

    cum = mask
    k = 1
    while k < BLOCK_T:
        z = jnp.zeros((NUM_EXPERTS, k), jnp.float32)
        cum = cum + jnp.concatenate([z, cum[:, :-k]], axis=1)
        k *= 2
    cum = cum - mask
    loc_in = jnp.sum(cum * mask, axis=0)
    offset = jnp.sum(cnt_ref[...] * mask, axis=0)
    loc_ref[...] = (loc_in + offset).astype(jnp.int32)

    cnt_ref[...] = cnt_ref[...] + blk_cnt

    @pl.when(pid == nblk - 1)
    def _fin():
        prod = me_ref[...] * cnt_ref[...]
        laux_ref[...] = jnp.sum(prod, axis=0, keepdims=True) * (
            NUM_EXPERTS / (NUM_TOKENS * NUM_TOKENS))


@jax.jit
def _top1_gate(x, W):
    nblk = NUM_TOKENS // BLOCK_T
    out_shapes = (
        jax.ShapeDtypeStruct((NUM_TOKENS,), jnp.int32),
        jax.ShapeDtypeStruct((NUM_TOKENS,), jnp.int32),
        jax.ShapeDtypeStruct((NUM_TOKENS,), jnp.float32),
        jax.ShapeDtypeStruct((1, 1), jnp.float32),
    )
    out = pl.pallas_call(
        _gate_body,
        grid=(nblk,),
        in_specs=[
            pl.BlockSpec((BLOCK_T // 2, MODEL_DIM), lambda i: (2 * i, 0)),
            pl.BlockSpec((BLOCK_T // 2, MODEL_DIM), lambda i: (2 * i + 1, 0)),
            pl.BlockSpec((NUM_EXPERTS, MODEL_DIM), lambda i: (0, 0)),
        ],
        out_specs=(
            pl.BlockSpec((BLOCK_T,), lambda i: (i,)),
            pl.BlockSpec((BLOCK_T,), lambda i: (i,)),
            pl.BlockSpec((BLOCK_T,), lambda i: (i,)),
            pl.BlockSpec((1, 1), lambda i: (0, 0)),
        ),
        out_shape=out_shapes,
        scratch_shapes=[
            pltpu.VMEM((NUM_EXPERTS, 1), jnp.float32),
            pltpu.VMEM((NUM_EXPERTS, 1), jnp.float32),
        ],
        compiler_params=pltpu.CompilerParams(
            dimension_semantics=("arbitrary",),
        ),
    )(x, x, W)
    idx, loc, gates1, laux = out
    return laux[0, 0], idx, loc, gates1


def kernel(x, W):
    laux, idx, loc, gates1 = _top1_gate(x, W)
    capacity = (NUM_TOKENS + NUM_EXPERTS - 1) // NUM_EXPERTS
    return (laux, idx, capacity, loc, gates1, NUM_EXPERTS)


# speedup vs baseline: 1.6924x; 1.0068x over previous
import jax
import jax.numpy as jnp
from jax import lax
from jax.experimental import pallas as pl
from jax.experimental.pallas import tpu as pltpu

MODEL_DIM = 2048
NUM_EXPERTS = 16
NUM_TOKENS = 8192
BLOCK_T = 1024


def _gate_body(x1_ref, x2_ref, w_ref, idx_ref, loc_ref, gate_ref, laux_ref,
               cnt_ref, me_ref):
    pid = pl.program_id(0)
    nblk = pl.num_programs(0)

    @pl.when(pid == 0)
    def _init():
        cnt_ref[...] = jnp.zeros_like(cnt_ref)
        me_ref[...] = jnp.zeros_like(me_ref)

    w = w_ref[...]
    lg1 = lax.dot_general(w, x1_ref[...], (((1,), (1,)), ((), ())),
                          preferred_element_type=jnp.float32)
    lg2 = lax.dot_general(w, x2_ref[...], (((1,), (1,)), ((), ())),
                          preferred_element_type=jnp.float32)
    lg = jnp.concatenate([lg1, lg2], axis=1)

    m = jnp.max(lg, axis=0, keepdims=True)
    p = jnp.exp(lg - m)
    s = jnp.sum(p, axis=0, keepdims=True)
    inv_s = 1.0 / s
    gate_ref[...] = inv_s[0]

    si = lax.broadcasted_iota(jnp.int32, lg.shape, 0)
    eq = lg == m
    idx = jnp.min(jnp.where(eq, si, NUM_EXPERTS), axis=0)
    idx_ref[...] = idx.astype(jnp.int32)

    mask = (si == idx[None, :]).astype(jnp.float32)

    me_ref[...] = me_ref[...] + jnp.sum(p * inv_s, axis=1, keepdims=True)
    blk_cnt = jnp.sum(mask, axis=1, keepdims=True)

    cum = mask
    k = 1
    while k < BLOCK_T:
        z = jnp.zeros((NUM_EXPERTS, k), jnp.float32)
        cum = cum + jnp.concatenate([z, cum[:, :-k]], axis=1)
        k *= 2
    cum = cum - mask
    loc_in = jnp.sum(cum * mask, axis=0)
    offset = jnp.sum(cnt_ref[...] * mask, axis=0)
    loc_ref[...] = (loc_in + offset).astype(jnp.int32)

    cnt_ref[...] = cnt_ref[...] + blk_cnt

    @pl.when(pid == nblk - 1)
    def _fin():
        prod = me_ref[...] * cnt_ref[...]
        laux_ref[...] = jnp.sum(prod, axis=0, keepdims=True) * (
            NUM_EXPERTS / (NUM_TOKENS * NUM_TOKENS))


@jax.jit
def _top1_gate(x, W):
    nblk = NUM_TOKENS // BLOCK_T
    out_shapes = (
        jax.ShapeDtypeStruct((NUM_TOKENS,), jnp.int32),
        jax.ShapeDtypeStruct((NUM_TOKENS,), jnp.int32),
        jax.ShapeDtypeStruct((NUM_TOKENS,), jnp.float32),
        jax.ShapeDtypeStruct((1, 1), jnp.float32),
    )
    out = pl.pallas_call(
        _gate_body,
        grid=(nblk,),
        in_specs=[
            pl.BlockSpec((BLOCK_T // 2, MODEL_DIM), lambda i: (2 * i, 0)),
            pl.BlockSpec((BLOCK_T // 2, MODEL_DIM), lambda i: (2 * i + 1, 0)),
            pl.BlockSpec((NUM_EXPERTS, MODEL_DIM), lambda i: (0, 0)),
        ],
        out_specs=(
            pl.BlockSpec((BLOCK_T,), lambda i: (i,)),
            pl.BlockSpec((BLOCK_T,), lambda i: (i,)),
            pl.BlockSpec((BLOCK_T,), lambda i: (i,)),
            pl.BlockSpec((1, 1), lambda i: (0, 0)),
        ),
        out_shape=out_shapes,
        scratch_shapes=[
            pltpu.VMEM((NUM_EXPERTS, 1), jnp.float32),
            pltpu.VMEM((NUM_EXPERTS, 1), jnp.float32),
        ],
        compiler_params=pltpu.CompilerParams(
            dimension_semantics=("arbitrary",),
        ),
    )(x, x, W)
    idx, loc, gates1, laux = out
    return laux[0, 0], idx, loc, gates1


def kernel(x, W):
    laux, idx, loc, gates1 = _top1_gate(x, W)
    capacity = (NUM_TOKENS + NUM_EXPERTS - 1) // NUM_EXPERTS
    return (laux, idx, capacity, loc, gates1, NUM_EXPERTS)
